# initial kernel scaffold (unmeasured)
import jax
import jax.numpy as jnp
from jax import lax
from jax.experimental import pallas as pl
from jax.experimental.pallas import tpu as pltpu

DBLK = 1024
F = 8192
OUT_ROWS = 2048


def kernel(x, dy):
    yi = lax.axis_index("y")
    zi = lax.axis_index("z")
    q = 2 * yi + zi
    x_sel = lax.dynamic_slice_in_dim(x, q * DBLK, DBLK, axis=1)
    p = lax.dot_general(
        x_sel.astype(jnp.bfloat16),
        dy.astype(jnp.bfloat16),
        dimension_numbers=(((0,), (0,)), ((), ())),
        preferred_element_type=jnp.float32,
    ).astype(jnp.bfloat16)
    out, _ = _comm(p)
    return out


def _comm(p):
    def body(p_ref, out_ref, prec_ref, acc_a, acc_b, ssem, rsem, csem):
        xi = lax.axis_index("x")
        yi = lax.axis_index("y")
        zi = lax.axis_index("z")
        is_owner = yi == xi
        x_peer = (1 - xi, yi, zi)
        y_peer = (xi, 1 - yi, zi)
        z_peer = (xi, yi, 1 - zi)
        my_rows = pl.ds(zi * DBLK, DBLK)
        other_rows = pl.ds((1 - zi) * DBLK, DBLK)

        p1 = pltpu.make_async_remote_copy(
            src_ref=p_ref,
            dst_ref=prec_ref,
            send_sem=ssem.at[0],
            recv_sem=rsem.at[0],
            device_id=x_peer,
            device_id_type=pl.DeviceIdType.MESH,
        )

        @pl.when(jnp.logical_not(is_owner))
        def _():
            p1.start()
            p1.wait_send()
            recv_y = pltpu.make_async_remote_copy(
                src_ref=out_ref.at[my_rows, :],
                dst_ref=out_ref.at[my_rows, :],
                send_sem=ssem.at[2],
                recv_sem=rsem.at[2],
                device_id=y_peer,
                device_id_type=pl.DeviceIdType.MESH,
            )
            recv_y.wait_recv()

        @pl.when(is_owner)
        def _():
            p1.wait_recv()
            cpa = pltpu.make_async_copy(p_ref, acc_a, csem.at[0])
            cpb = pltpu.make_async_copy(prec_ref, acc_b, csem.at[1])
            cpa.start()
            cpb.start()
            cpa.wait()
            cpb.wait()
            acc_a[...] = (
                acc_a[...].astype(jnp.float32) + acc_b[...].astype(jnp.float32)
            ).astype(jnp.bfloat16)
            cpo = pltpu.make_async_copy(acc_a, out_ref.at[my_rows, :], csem.at[0])
            cpo.start()
            cpo.wait()
            send_y = pltpu.make_async_remote_copy(
                src_ref=out_ref.at[my_rows, :],
                dst_ref=out_ref.at[my_rows, :],
                send_sem=ssem.at[2],
                recv_sem=rsem.at[2],
                device_id=y_peer,
                device_id_type=pl.DeviceIdType.MESH,
            )
            send_y.start()
            send_y.wait_send()

        exz = pltpu.make_async_remote_copy(
            src_ref=out_ref.at[my_rows, :],
            dst_ref=out_ref.at[my_rows, :],
            send_sem=ssem.at[1],
            recv_sem=rsem.at[1],
            device_id=z_peer,
            device_id_type=pl.DeviceIdType.MESH,
        )
        exz.start()
        exz.wait_send()
        recv_z = pltpu.make_async_remote_copy(
            src_ref=out_ref.at[other_rows, :],
            dst_ref=out_ref.at[other_rows, :],
            send_sem=ssem.at[1],
            recv_sem=rsem.at[1],
            device_id=z_peer,
            device_id_type=pl.DeviceIdType.MESH,
        )
        recv_z.wait_recv()

    out, _prec = pl.pallas_call(
        body,
        out_shape=(
            jax.ShapeDtypeStruct((OUT_ROWS, F), jnp.bfloat16),
            jax.ShapeDtypeStruct((DBLK, F), jnp.bfloat16),
        ),
        in_specs=[pl.BlockSpec(memory_space=pltpu.ANY)],
        out_specs=(
            pl.BlockSpec(memory_space=pltpu.ANY),
            pl.BlockSpec(memory_space=pltpu.ANY),
        ),
        scratch_shapes=[
            pltpu.VMEM((DBLK, F), jnp.bfloat16),
            pltpu.VMEM((DBLK, F), jnp.bfloat16),
            pltpu.SemaphoreType.DMA((3,)),
            pltpu.SemaphoreType.DMA((3,)),
            pltpu.SemaphoreType.DMA((2,)),
        ],
    )(p)
    return out, _prec


# baseline (device time: 701475 ns/iter reference)
import jax
import jax.numpy as jnp
from jax import lax
from jax.experimental import pallas as pl
from jax.experimental.pallas import tpu as pltpu

DBLK = 1024
F = 8192
OUT_ROWS = 2048


def kernel(x, dy):
    yi = lax.axis_index("y")
    zi = lax.axis_index("z")
    q = 2 * yi + zi
    x_sel = lax.dynamic_slice_in_dim(x, q * DBLK, DBLK, axis=1)
    p = lax.dot_general(
        x_sel.astype(jnp.bfloat16),
        dy.astype(jnp.bfloat16),
        dimension_numbers=(((0,), (0,)), ((), ())),
        preferred_element_type=jnp.float32,
    ).astype(jnp.bfloat16)
    out, _ = _comm(p)
    return out


def _comm(p):
    def body(p_ref, out_ref, prec_ref, acc_a, acc_b, ssem, rsem, csem):
        xi = lax.axis_index("x")
        yi = lax.axis_index("y")
        zi = lax.axis_index("z")
        is_owner = yi == xi
        x_peer = (1 - xi, yi, zi)
        y_peer = (xi, 1 - yi, zi)
        z_peer = (xi, yi, 1 - zi)
        my_rows = pl.ds(zi * DBLK, DBLK)
        other_rows = pl.ds((1 - zi) * DBLK, DBLK)

        p1 = pltpu.make_async_remote_copy(
            src_ref=p_ref,
            dst_ref=prec_ref,
            send_sem=ssem.at[0],
            recv_sem=rsem.at[0],
            device_id=x_peer,
            device_id_type=pl.DeviceIdType.MESH,
        )

        @pl.when(jnp.logical_not(is_owner))
        def _():
            p1.start()
            p1.wait_send()
            recv_y = pltpu.make_async_remote_copy(
                src_ref=out_ref.at[my_rows, :],
                dst_ref=out_ref.at[my_rows, :],
                send_sem=ssem.at[2],
                recv_sem=rsem.at[2],
                device_id=y_peer,
                device_id_type=pl.DeviceIdType.MESH,
            )
            recv_y.wait_recv()

        @pl.when(is_owner)
        def _():
            p1.wait_recv()
            cpa = pltpu.make_async_copy(p_ref, acc_a, csem.at[0])
            cpb = pltpu.make_async_copy(prec_ref, acc_b, csem.at[1])
            cpa.start()
            cpb.start()
            cpa.wait()
            cpb.wait()
            acc_a[...] = (
                acc_a[...].astype(jnp.float32) + acc_b[...].astype(jnp.float32)
            ).astype(jnp.bfloat16)
            cpo = pltpu.make_async_copy(acc_a, out_ref.at[my_rows, :], csem.at[0])
            cpo.start()
            cpo.wait()
            send_y = pltpu.make_async_remote_copy(
                src_ref=out_ref.at[my_rows, :],
                dst_ref=out_ref.at[my_rows, :],
                send_sem=ssem.at[2],
                recv_sem=rsem.at[2],
                device_id=y_peer,
                device_id_type=pl.DeviceIdType.MESH,
            )
            send_y.start()
            send_y.wait_send()

        exz = pltpu.make_async_remote_copy(
            src_ref=out_ref.at[my_rows, :],
            dst_ref=out_ref.at[my_rows, :],
            send_sem=ssem.at[1],
            recv_sem=rsem.at[1],
            device_id=z_peer,
            device_id_type=pl.DeviceIdType.MESH,
        )
        exz.start()
        exz.wait_send()
        recv_z = pltpu.make_async_remote_copy(
            src_ref=out_ref.at[other_rows, :],
            dst_ref=out_ref.at[other_rows, :],
            send_sem=ssem.at[1],
            recv_sem=rsem.at[1],
            device_id=z_peer,
            device_id_type=pl.DeviceIdType.MESH,
        )
        recv_z.wait_recv()

    out, _prec = pl.pallas_call(
        body,
        out_shape=(
            jax.ShapeDtypeStruct((OUT_ROWS, F), jnp.bfloat16),
            jax.ShapeDtypeStruct((DBLK, F), jnp.bfloat16),
        ),
        in_specs=[pl.BlockSpec(memory_space=pl.ANY)],
        out_specs=(
            pl.BlockSpec(memory_space=pl.ANY),
            pl.BlockSpec(memory_space=pl.ANY),
        ),
        scratch_shapes=[
            pltpu.VMEM((DBLK, F), jnp.bfloat16),
            pltpu.VMEM((DBLK, F), jnp.bfloat16),
            pltpu.SemaphoreType.DMA((3,)),
            pltpu.SemaphoreType.DMA((3,)),
            pltpu.SemaphoreType.DMA((2,)),
        ],
    )(p)
    return out, _prec
